# baseline (speedup 1.0000x reference)
import jax
import jax.numpy as jnp
import numpy as np
from jax import lax
from jax.experimental import pallas as pl
from jax.experimental.pallas import tpu as pltpu
from jax.experimental.pallas import tpu_sc as plsc

VOCAB = 1000000
WDIM = 64
PDIM = 50
ODIM = WDIM + PDIM
OPAD = 120
DPAD = 56
NDIST = 104
MAXLEN = 31
B = 16384
TOK = B * MAXLEN

NC, NS, L = 2, 16, 16
NW = NC * NS

ROWS_W = B // NW
ROWS_C = 16
CHUNKS = ROWS_W // ROWS_C
C = ROWS_C * MAXLEN
GSUB = 128


def _body(idx_hbm, dst_hbm, len_hbm, word_hbm, dt_hbm, zc_hbm, out_hbm,
          idx_v, dst_v, len_v, zbuf, dt_sh, wslab, dslab,
          sem_w, sem_d):
    sid = lax.axis_index("s")
    wid = sid * NC + lax.axis_index("c")
    iota = lax.iota(jnp.int32, L)
    zero_i = jnp.zeros((L,), jnp.int32)
    zrow_i = jnp.full((L,), VOCAB, jnp.int32)
    pltpu.sync_copy(zc_hbm, zbuf)
    rtzero = zbuf[:]

    @pl.when(sid == 0)
    def _stage():
        pltpu.sync_copy(dt_hbm, dt_sh)

    plsc.subcore_barrier()

    @pl.loop(0, CHUNKS)
    def _chunk(c):
        rowbase = wid * ROWS_W + c * ROWS_C
        tokbase = rowbase * MAXLEN

        pltpu.sync_copy(idx_hbm.at[pl.ds(tokbase, C)], idx_v)
        pltpu.sync_copy(dst_hbm.at[pl.ds(tokbase, C)], dst_v)
        pltpu.sync_copy(len_hbm.at[pl.ds(rowbase, ROWS_C)], len_v)

        for g in range(C // L):
            e = jnp.full((L,), g * L, jnp.int32) + iota
            brow = lax.shift_right_logical(
                e * jnp.full((L,), 529, jnp.int32),
                jnp.full((L,), 14, jnp.int32)) + rtzero
            pos = e - brow * jnp.full((L,), MAXLEN, jnp.int32)
            lv = plsc.load_gather(len_v, [brow])
            msk = pos < lv
            icur = idx_v[pl.ds(g * L, L)]
            idx_v[pl.ds(g * L, L)] = jnp.where(msk, icur, zrow_i)
            dcur = dst_v[pl.ds(g * L, L)]
            dst_v[pl.ds(g * L, L)] = jnp.where(msk, dcur, zero_i)

        copies = []
        off = 0
        while off < C:
            n = min(GSUB, C - off)
            copies.append(pltpu.async_copy(
                word_hbm.at[idx_v.at[pl.ds(off, n)]],
                wslab.at[pl.ds(off, n)], sem_w))
            copies.append(pltpu.async_copy(
                dt_sh.at[dst_v.at[pl.ds(off, n)]],
                dslab.at[pl.ds(off, n)], sem_d))
            off += n
        for cp in copies:
            cp.wait()

        pltpu.sync_copy(wslab, out_hbm.at[pl.ds(tokbase, C), pl.ds(0, WDIM)])
        pltpu.sync_copy(dslab,
                        out_hbm.at[pl.ds(tokbase, C), pl.ds(WDIM, DPAD)])


@jax.jit
def _run(idx_f, dst_f, length, wext, dt0):
    mesh = plsc.VectorSubcoreMesh(core_axis_name="c", subcore_axis_name="s")
    return pl.kernel(
        _body,
        out_type=jax.ShapeDtypeStruct((TOK, OPAD), jnp.float32),
        mesh=mesh,
        compiler_params=pltpu.CompilerParams(
            needs_layout_passes=False, use_tc_tiling_on_sc=False),
        scratch_types=[
            pltpu.VMEM((C,), jnp.int32),
            pltpu.VMEM((C,), jnp.int32),
            pltpu.VMEM((ROWS_C,), jnp.int32),
            pltpu.VMEM((L,), jnp.int32),
            pltpu.VMEM_SHARED((NDIST, DPAD), jnp.float32),
            pltpu.VMEM((C, WDIM), jnp.float32),
            pltpu.VMEM((C, DPAD), jnp.float32),
            pltpu.SemaphoreType.DMA,
            pltpu.SemaphoreType.DMA,
        ],
    )(idx_f, dst_f, length, wext, dt0, jnp.zeros((L,), jnp.int32))


def kernel(indices, dist, length, word_table, dist_table):
    wext = jnp.concatenate(
        [word_table, jnp.zeros((8, WDIM), word_table.dtype)], axis=0)
    dt0 = jnp.zeros((NDIST, DPAD), dist_table.dtype)
    dt0 = dt0.at[:dist_table.shape[0], :PDIM].set(dist_table).at[0].set(0.0)
    out = _run(indices.reshape(-1), dist.reshape(-1), length.reshape(-1),
               wext, dt0)
    return out[:, :ODIM].reshape(B, MAXLEN, ODIM)


# speedup vs baseline: 1.0015x; 1.0015x over previous
import jax
import jax.numpy as jnp
import numpy as np
from jax import lax
from jax.experimental import pallas as pl
from jax.experimental.pallas import tpu as pltpu
from jax.experimental.pallas import tpu_sc as plsc

VOCAB = 1000000
WDIM = 64
PDIM = 50
ODIM = WDIM + PDIM
OPAD = 120
DPAD = 56
NDIST = 104
MAXLEN = 31
B = 16384
TOK = B * MAXLEN

NC, NS, L = 2, 16, 16
NW = NC * NS

ROWS_W = B // NW
ROWS_C = 16
CHUNKS = ROWS_W // ROWS_C
C = ROWS_C * MAXLEN
GSUB = 128


def _body(idx_hbm, dst_hbm, len_hbm, word_hbm, dt_hbm, zc_hbm, out_hbm,
          idx_v, dst_v, len_v, zbuf, dt_sh, wslab, dslab,
          sem_w, sem_d):
    sid = lax.axis_index("s")
    wid = sid * NC + lax.axis_index("c")
    iota = lax.iota(jnp.int32, L)
    zero_i = jnp.zeros((L,), jnp.int32)
    zrow_i = jnp.full((L,), VOCAB, jnp.int32)
    pltpu.sync_copy(zc_hbm, zbuf)
    rtzero = zbuf[:]

    @pl.when(sid == 0)
    def _stage():
        pltpu.sync_copy(dt_hbm, dt_sh)

    plsc.subcore_barrier()

    @pl.loop(0, CHUNKS)
    def _chunk(c):
        rowbase = wid * ROWS_W + c * ROWS_C
        tokbase = rowbase * MAXLEN

        pltpu.sync_copy(idx_hbm.at[pl.ds(tokbase, C)], idx_v)
        pltpu.sync_copy(dst_hbm.at[pl.ds(tokbase, C)], dst_v)
        pltpu.sync_copy(len_hbm.at[pl.ds(rowbase, ROWS_C)], len_v)

        @pl.loop(0, C // L)
        def _grp(g):
            o = g * L
            e = lax.broadcast_in_dim(o.astype(jnp.int32), (L,), ()) + iota
            brow = lax.shift_right_logical(
                e * jnp.full((L,), 529, jnp.int32),
                jnp.full((L,), 14, jnp.int32)) + rtzero
            pos = e - brow * jnp.full((L,), MAXLEN, jnp.int32)
            lv = plsc.load_gather(len_v, [brow])
            msk = pos < lv
            icur = idx_v[pl.ds(o, L)]
            idx_v[pl.ds(o, L)] = jnp.where(msk, icur, zrow_i)
            dcur = dst_v[pl.ds(o, L)]
            dst_v[pl.ds(o, L)] = jnp.where(msk, dcur, zero_i)

        copies = []
        off = 0
        while off < C:
            n = min(GSUB, C - off)
            copies.append(pltpu.async_copy(
                word_hbm.at[idx_v.at[pl.ds(off, n)]],
                wslab.at[pl.ds(off, n)], sem_w))
            copies.append(pltpu.async_copy(
                dt_sh.at[dst_v.at[pl.ds(off, n)]],
                dslab.at[pl.ds(off, n)], sem_d))
            off += n
        for cp in copies:
            cp.wait()

        pltpu.sync_copy(wslab, out_hbm.at[pl.ds(tokbase, C), pl.ds(0, WDIM)])
        pltpu.sync_copy(dslab,
                        out_hbm.at[pl.ds(tokbase, C), pl.ds(WDIM, DPAD)])


@jax.jit
def _run(idx_f, dst_f, length, wext, dt0):
    mesh = plsc.VectorSubcoreMesh(core_axis_name="c", subcore_axis_name="s")
    return pl.kernel(
        _body,
        out_type=jax.ShapeDtypeStruct((TOK, OPAD), jnp.float32),
        mesh=mesh,
        compiler_params=pltpu.CompilerParams(
            needs_layout_passes=False, use_tc_tiling_on_sc=False),
        scratch_types=[
            pltpu.VMEM((C,), jnp.int32),
            pltpu.VMEM((C,), jnp.int32),
            pltpu.VMEM((ROWS_C,), jnp.int32),
            pltpu.VMEM((L,), jnp.int32),
            pltpu.VMEM_SHARED((NDIST, DPAD), jnp.float32),
            pltpu.VMEM((C, WDIM), jnp.float32),
            pltpu.VMEM((C, DPAD), jnp.float32),
            pltpu.SemaphoreType.DMA,
            pltpu.SemaphoreType.DMA,
        ],
    )(idx_f, dst_f, length, wext, dt0, jnp.zeros((L,), jnp.int32))


def kernel(indices, dist, length, word_table, dist_table):
    wext = jnp.concatenate(
        [word_table, jnp.zeros((8, WDIM), word_table.dtype)], axis=0)
    dt0 = jnp.zeros((NDIST, DPAD), dist_table.dtype)
    dt0 = dt0.at[:dist_table.shape[0], :PDIM].set(dist_table).at[0].set(0.0)
    out = _run(indices.reshape(-1), dist.reshape(-1), length.reshape(-1),
               wext, dt0)
    return out[:, :ODIM].reshape(B, MAXLEN, ODIM)


# speedup vs baseline: 1.0030x; 1.0015x over previous
import jax
import jax.numpy as jnp
import numpy as np
from jax import lax
from jax.experimental import pallas as pl
from jax.experimental.pallas import tpu as pltpu
from jax.experimental.pallas import tpu_sc as plsc

VOCAB = 1000000
WDIM = 64
PDIM = 50
ODIM = WDIM + PDIM
OPAD = 120
DPAD = 56
NDIST = 104
MAXLEN = 31
B = 16384
TOK = B * MAXLEN

NC, NS, L = 2, 16, 16
NW = NC * NS

ROWS_W = B // NW
ROWS_C = 16
CHUNKS = ROWS_W // ROWS_C
C = ROWS_C * MAXLEN
GSUB = 128


def _body(idx_hbm, dst_hbm, len_hbm, word_hbm, dt_hbm, out_hbm,
          idx_v, dst_v, len_v, dt_sh, wslab, dslab,
          sem_w, sem_d):
    sid = lax.axis_index("s")
    wid = sid * NC + lax.axis_index("c")
    iota = lax.iota(jnp.int32, L)
    zero_i = jnp.zeros((L,), jnp.int32)
    zrow_i = jnp.full((L,), VOCAB, jnp.int32)

    @pl.when(sid == 0)
    def _stage():
        pltpu.sync_copy(dt_hbm, dt_sh)

    plsc.subcore_barrier()

    @pl.loop(0, CHUNKS)
    def _chunk(c):
        rowbase = wid * ROWS_W + c * ROWS_C
        tokbase = rowbase * MAXLEN

        pltpu.sync_copy(idx_hbm.at[pl.ds(tokbase, C)], idx_v)
        pltpu.sync_copy(dst_hbm.at[pl.ds(tokbase, C)], dst_v)
        pltpu.sync_copy(len_hbm.at[pl.ds(tokbase, C)], len_v)

        @pl.loop(0, C // L)
        def _grp(g):
            o = g * L
            e = lax.broadcast_in_dim(o.astype(jnp.int32), (L,), ()) + iota
            brow = lax.shift_right_logical(
                e * jnp.full((L,), 529, jnp.int32),
                jnp.full((L,), 14, jnp.int32))
            pos = e - brow * jnp.full((L,), MAXLEN, jnp.int32)
            lv = len_v[pl.ds(o, L)]
            msk = pos < lv
            icur = idx_v[pl.ds(o, L)]
            idx_v[pl.ds(o, L)] = jnp.where(msk, icur, zrow_i)
            dcur = dst_v[pl.ds(o, L)]
            dst_v[pl.ds(o, L)] = jnp.where(msk, dcur, zero_i)

        copies = []
        off = 0
        while off < C:
            n = min(GSUB, C - off)
            copies.append(pltpu.async_copy(
                word_hbm.at[idx_v.at[pl.ds(off, n)]],
                wslab.at[pl.ds(off, n)], sem_w))
            copies.append(pltpu.async_copy(
                dt_sh.at[dst_v.at[pl.ds(off, n)]],
                dslab.at[pl.ds(off, n)], sem_d))
            off += n
        for cp in copies:
            cp.wait()

        pltpu.sync_copy(wslab, out_hbm.at[pl.ds(tokbase, C), pl.ds(0, WDIM)])
        pltpu.sync_copy(dslab,
                        out_hbm.at[pl.ds(tokbase, C), pl.ds(WDIM, DPAD)])


@jax.jit
def _run(idx_f, dst_f, length, wext, dt0):
    mesh = plsc.VectorSubcoreMesh(core_axis_name="c", subcore_axis_name="s")
    return pl.kernel(
        _body,
        out_type=jax.ShapeDtypeStruct((TOK, OPAD), jnp.float32),
        mesh=mesh,
        compiler_params=pltpu.CompilerParams(
            needs_layout_passes=False, use_tc_tiling_on_sc=False),
        scratch_types=[
            pltpu.VMEM((C,), jnp.int32),
            pltpu.VMEM((C,), jnp.int32),
            pltpu.VMEM((C,), jnp.int32),
            pltpu.VMEM_SHARED((NDIST, DPAD), jnp.float32),
            pltpu.VMEM((C, WDIM), jnp.float32),
            pltpu.VMEM((C, DPAD), jnp.float32),
            pltpu.SemaphoreType.DMA,
            pltpu.SemaphoreType.DMA,
        ],
    )(idx_f, dst_f, length, wext, dt0)


def kernel(indices, dist, length, word_table, dist_table):
    wext = jnp.concatenate(
        [word_table, jnp.zeros((8, WDIM), word_table.dtype)], axis=0)
    dt0 = jnp.zeros((NDIST, DPAD), dist_table.dtype)
    dt0 = dt0.at[:dist_table.shape[0], :PDIM].set(dist_table).at[0].set(0.0)
    lenr = jnp.repeat(length.reshape(-1), MAXLEN)
    out = _run(indices.reshape(-1), dist.reshape(-1), lenr, wext, dt0)
    return out[:, :ODIM].reshape(B, MAXLEN, ODIM)
